# baseline (speedup 1.0000x reference)
import functools

import jax
import jax.numpy as jnp
from jax import lax
from jax.experimental import pallas as pl
from jax.experimental.pallas import tpu as pltpu
from jax.experimental.pallas import tpu_sc as plsc

N = 10000
E = 320000
T = 128
H = 512
L = 128

NPAD = 10240
BLK = 512
NBLK = NPAD // BLK

NC = 2
NS = 16
NW = NC * NS
EPW = E // NW
KCH = 80
NCHUNK = EPW // KCH
ZR = 8
RPT = NPAD // NS
KC2 = 125
NC2 = EPW // KC2


def _seg_mesh():
    return plsc.VectorSubcoreMesh(core_axis_name="c", subcore_axis_name="s")


KW = 128
NWCH = EPW // KW
TAIL = EPW - NWCH * KW
assert NWCH % 4 == 2 and TAIL % 8 == 0


@functools.partial(
    pl.kernel,
    out_type=jax.ShapeDtypeStruct((NC, NPAD, L), jnp.float32),
    mesh=plsc.VectorSubcoreMesh(core_axis_name="c", subcore_axis_name="s"),
    scratch_types=[
        pltpu.VMEM((4, KW), jnp.int32),
        pltpu.VMEM((4, KW), jnp.int32),
        pltpu.VMEM((TAIL,), jnp.int32),
        pltpu.VMEM((TAIL,), jnp.int32),
        pltpu.VMEM((KW, L), jnp.float32),
        pltpu.VMEM((KW, L), jnp.float32),
        pltpu.VMEM((ZR, L), jnp.float32),
        pltpu.VMEM_SHARED((NPAD, L), jnp.float32),
        pltpu.SemaphoreType.DMA,
        pltpu.SemaphoreType.DMA,
        pltpu.SemaphoreType.DMA,
        pltpu.SemaphoreType.DMA,
        pltpu.SemaphoreType.DMA,
        pltpu.SemaphoreType.DMA,
        pltpu.SemaphoreType.DMA,
        pltpu.SemaphoreType.DMA,
        pltpu.SemaphoreType.DMA,
        pltpu.SemaphoreType.DMA,
    ],
)
def _segsum_wide(x_hbm, src_hbm, dst_hbm, out_hbm, sidx, didx, tsi, tdi,
                 rows0, rows1, zv, acc,
                 ss0, ss1, ss2, ss3, ds0, ds1, ds2, ds3, g0, g1):
    c = lax.axis_index("c")
    s = lax.axis_index("s")
    wid = s * NC + c
    ebase = wid * EPW
    ssem = (ss0, ss1, ss2, ss3)
    dsem = (ds0, ds1, ds2, ds3)
    rows = (rows0, rows1)
    gsem = (g0, g1)

    def load_idx(j, p):
        off = ebase + j * KW
        pltpu.async_copy(src_hbm.at[pl.ds(off, KW)], sidx.at[p], ssem[p])
        pltpu.async_copy(dst_hbm.at[pl.ds(off, KW)], didx.at[p], dsem[p])

    def wait_s(p):
        pltpu.make_async_copy(src_hbm.at[pl.ds(0, KW)], sidx.at[p],
                              ssem[p]).wait()

    def start(p, b):
        pltpu.async_copy(x_hbm.at[sidx.at[p]], rows[b], gsem[b])

    def finish(p, b):
        pltpu.make_async_copy(x_hbm.at[sidx.at[p]], rows[b], gsem[b]).wait()
        pltpu.make_async_copy(dst_hbm.at[pl.ds(0, KW)], didx.at[p],
                              dsem[p]).wait()
        pltpu.sync_copy(rows[b], acc.at[didx.at[p]], add=True)

    load_idx(0, 0)
    load_idx(1, 1)
    load_idx(2, 2)
    load_idx(3, 3)

    z16 = jnp.zeros((16,), jnp.float32)

    def zero_block(i, carry):
        zv[i // 8, pl.ds((i % 8) * 16, 16)] = z16
        return carry

    lax.fori_loop(0, ZR * 8, zero_block, 0)

    row0 = s * RPT

    def zero_acc(j, carry):
        pltpu.sync_copy(zv, acc.at[pl.ds(row0 + j * ZR, ZR)])
        return carry

    lax.fori_loop(0, RPT // ZR, zero_acc, 0)
    plsc.subcore_barrier()

    wait_s(0)
    start(0, 0)

    def quad(jo, carry):
        j = jo * 4
        wait_s(1)
        start(1, 1)
        finish(0, 0)
        load_idx(j + 4, 0)
        wait_s(2)
        start(2, 0)
        finish(1, 1)

        @pl.when(j + 5 < NWCH)
        def _():
            load_idx(j + 5, 1)

        wait_s(3)
        start(3, 1)
        finish(2, 0)

        @pl.when(j + 6 < NWCH)
        def _():
            load_idx(j + 6, 2)

        wait_s(0)
        start(0, 0)
        finish(3, 1)

        @pl.when(j + 7 < NWCH)
        def _():
            load_idx(j + 7, 3)

        return carry

    lax.fori_loop(0, (NWCH - 2) // 4, quad, 0)
    wait_s(1)
    start(1, 1)
    finish(0, 0)
    finish(1, 1)

    toff = ebase + NWCH * KW
    pltpu.sync_copy(src_hbm.at[pl.ds(toff, TAIL)], tsi)
    pltpu.sync_copy(dst_hbm.at[pl.ds(toff, TAIL)], tdi)
    pltpu.async_copy(x_hbm.at[tsi], rows0.at[pl.ds(0, TAIL)], g0).wait()
    pltpu.sync_copy(rows0.at[pl.ds(0, TAIL)], acc.at[tdi], add=True)

    plsc.subcore_barrier()
    pltpu.sync_copy(acc.at[pl.ds(row0, RPT)], out_hbm.at[c, pl.ds(row0, RPT)])


@functools.partial(
    pl.kernel,
    out_type=jax.ShapeDtypeStruct((NC, NPAD), jnp.float32),
    mesh=plsc.VectorSubcoreMesh(core_axis_name="c", subcore_axis_name="s"),
    scratch_types=[
        pltpu.VMEM((NC2, KC2), jnp.int32),
        pltpu.VMEM((NC2, KC2), jnp.int32),
        pltpu.VMEM((KC2,), jnp.float32),
        pltpu.VMEM((KC2,), jnp.float32),
        pltpu.VMEM((RPT,), jnp.float32),
        pltpu.VMEM_SHARED((NPAD,), jnp.float32),
        pltpu.SemaphoreType.DMA,
        pltpu.SemaphoreType.DMA,
        pltpu.SemaphoreType.DMA,
    ],
)
def _segsum_scalar(x_hbm, src_hbm, dst_hbm, out_hbm, src2, dst2, rows0, rows1,
                   zv, acc, isem, sem0, sem1):
    c = lax.axis_index("c")
    s = lax.axis_index("s")
    wid = s * NC + c

    idx_cp0 = pltpu.async_copy(src_hbm.at[wid], src2, isem)
    idx_cp1 = pltpu.async_copy(dst_hbm.at[wid], dst2, isem)

    z16 = jnp.zeros((16,), jnp.float32)

    def zero_block(i, carry):
        zv[pl.ds(i * 16, 16)] = z16
        return carry

    lax.fori_loop(0, RPT // 16, zero_block, 0)

    row0 = s * RPT
    pltpu.sync_copy(zv, acc.at[pl.ds(row0, RPT)])
    idx_cp0.wait()
    idx_cp1.wait()
    plsc.subcore_barrier()

    rows = (rows0, rows1)
    sems = (sem0, sem1)

    def start(j, b):
        pltpu.async_copy(x_hbm.at[src2.at[j]], rows[b], sems[b])

    def finish(j, b):
        pltpu.make_async_copy(x_hbm.at[src2.at[j]], rows[b], sems[b]).wait()
        pltpu.sync_copy(rows[b], acc.at[dst2.at[j]], add=True)

    start(0, 0)

    def pair(jo, carry):
        j = jo * 2
        start(j + 1, 1)
        finish(j, 0)
        start(j + 2, 0)
        finish(j + 1, 1)
        return carry

    if NC2 % 2 == 0:
        lax.fori_loop(0, NC2 // 2 - 1, pair, 0)
        start(NC2 - 1, 1)
        finish(NC2 - 2, 0)
        finish(NC2 - 1, 1)
    else:
        lax.fori_loop(0, NC2 // 2, pair, 0)
        finish(NC2 - 1, 0)

    plsc.subcore_barrier()
    pltpu.sync_copy(acc.at[pl.ds(row0, RPT)], out_hbm.at[c, pl.ds(row0, RPT)])


def _dot(a, b):
    return jnp.dot(a, b, preferred_element_type=jnp.float32)


def _mlp_a_body(x_ref, p0_ref, p1_ref, wa1_ref, ba1_ref, wa2_ref, ba2_ref,
                wb1_ref, y2_ref):
    x = x_ref[...] + p0_ref[...] + p1_ref[...]
    h = jnp.maximum(_dot(x, wa1_ref[...]) + ba1_ref[...], 0.0)
    x1 = jnp.maximum(_dot(h, wa2_ref[...]) + ba2_ref[...], 0.0)
    y2_ref[...] = _dot(x1, wb1_ref[...])


_mlp_a = pl.pallas_call(
    _mlp_a_body,
    grid=(NBLK,),
    in_specs=[
        pl.BlockSpec((BLK, T), lambda i: (i, 0)),
        pl.BlockSpec((BLK, T), lambda i: (i, 0)),
        pl.BlockSpec((BLK, T), lambda i: (i, 0)),
        pl.BlockSpec((T, H), lambda i: (0, 0)),
        pl.BlockSpec((1, H), lambda i: (0, 0)),
        pl.BlockSpec((H, H), lambda i: (0, 0)),
        pl.BlockSpec((1, H), lambda i: (0, 0)),
        pl.BlockSpec((H, L), lambda i: (0, 0)),
    ],
    out_specs=pl.BlockSpec((BLK, L), lambda i: (i, 0)),
    out_shape=jax.ShapeDtypeStruct((NPAD, L), jnp.float32),
)


def _mlp_b_body(y2_ref, p0_ref, p1_ref, bb1_ref, wb2_ref, bb2_ref, wdem_ref,
                feat_ref, dem_ref):
    h = jnp.maximum(y2_ref[...] + p0_ref[...] + p1_ref[...] + bb1_ref[...], 0.0)
    feat = _dot(h, wb2_ref[...]) + bb2_ref[...]
    feat_ref[...] = feat

    @pl.when(pl.program_id(0) == 0)
    def _():
        dem_ref[...] = jnp.zeros_like(dem_ref)

    dem_ref[...] += jnp.sum(feat * wdem_ref[...]).reshape(1, 1)


_mlp_b = pl.pallas_call(
    _mlp_b_body,
    grid=(NBLK,),
    in_specs=[
        pl.BlockSpec((BLK, L), lambda i: (i, 0)),
        pl.BlockSpec((BLK, L), lambda i: (i, 0)),
        pl.BlockSpec((BLK, L), lambda i: (i, 0)),
        pl.BlockSpec((1, L), lambda i: (0, 0)),
        pl.BlockSpec((L, L), lambda i: (0, 0)),
        pl.BlockSpec((1, L), lambda i: (0, 0)),
        pl.BlockSpec((BLK, L), lambda i: (i, 0)),
    ],
    out_specs=[
        pl.BlockSpec((BLK, L), lambda i: (i, 0)),
        pl.BlockSpec((1, 1), lambda i: (0, 0)),
    ],
    out_shape=[
        jax.ShapeDtypeStruct((NPAD, L), jnp.float32),
        jax.ShapeDtypeStruct((1, 1), jnp.float32),
    ],
)


def _mlp_c_body(f_ref, p0_ref, p1_ref, wc1_ref, bc1_ref, wc2_ref, bc2_ref,
                wd1_ref, y4_ref):
    x = f_ref[...] + p0_ref[...] + p1_ref[...]
    h = jnp.maximum(_dot(x, wc1_ref[...]) + bc1_ref[...], 0.0)
    x3 = jnp.maximum(_dot(h, wc2_ref[...]) + bc2_ref[...], 0.0)
    y4_ref[...] = _dot(x3, wd1_ref[...])


_mlp_c = pl.pallas_call(
    _mlp_c_body,
    grid=(NBLK,),
    in_specs=[
        pl.BlockSpec((BLK, L), lambda i: (i, 0)),
        pl.BlockSpec((BLK, L), lambda i: (i, 0)),
        pl.BlockSpec((BLK, L), lambda i: (i, 0)),
        pl.BlockSpec((L, H), lambda i: (0, 0)),
        pl.BlockSpec((1, H), lambda i: (0, 0)),
        pl.BlockSpec((H, H), lambda i: (0, 0)),
        pl.BlockSpec((1, H), lambda i: (0, 0)),
        pl.BlockSpec((H, 1), lambda i: (0, 0)),
    ],
    out_specs=pl.BlockSpec((BLK, 1), lambda i: (i, 0)),
    out_shape=jax.ShapeDtypeStruct((NPAD, 1), jnp.float32),
)


def _final_body(y4_ref, q0_ref, q1_ref, bd1_ref, wd2_ref, bd2_ref,
                demraw_ref, bdem_ref, reg_ref, dem_ref):
    s4 = jnp.maximum(y4_ref[...] + q0_ref[...] + q1_ref[...] + bd1_ref[...], 0.0)
    reg_ref[...] = jax.nn.sigmoid(s4 * wd2_ref[...] + bd2_ref[...])
    dem_ref[...] = jax.nn.sigmoid(demraw_ref[...] + bdem_ref[...])


_R8 = NPAD // 128

_final = pl.pallas_call(
    _final_body,
    grid=(1,),
    in_specs=[
        pl.BlockSpec((_R8, 128), lambda i: (0, 0)),
        pl.BlockSpec((_R8, 128), lambda i: (0, 0)),
        pl.BlockSpec((_R8, 128), lambda i: (0, 0)),
        pl.BlockSpec((1, 1), lambda i: (0, 0)),
        pl.BlockSpec((1, 1), lambda i: (0, 0)),
        pl.BlockSpec((1, 1), lambda i: (0, 0)),
        pl.BlockSpec((1, 1), lambda i: (0, 0)),
        pl.BlockSpec((1, 1), lambda i: (0, 0)),
    ],
    out_specs=[
        pl.BlockSpec((_R8, 128), lambda i: (0, 0)),
        pl.BlockSpec((1, 1), lambda i: (0, 0)),
    ],
    out_shape=[
        jax.ShapeDtypeStruct((_R8, 128), jnp.float32),
        jax.ShapeDtypeStruct((1, 1), jnp.float32),
    ],
)


def kernel(eeg_nodes, eeg_idx, eeg_attr, Wa1, ba1, Wa2, ba2, Wb1, bb1, Wb2,
           bb2, Wc1, bc1, Wc2, bc2, Wd1, bd1, Wd2, bd2, Wdem, bdem):
    src = eeg_idx[0]
    dst = eeg_idx[1]
    src3 = src.reshape(NW, NC2, KC2)
    dst3 = dst.reshape(NW, NC2, KC2)

    x0 = jnp.pad(eeg_nodes, ((0, NPAD - N), (0, 0)))

    p1 = _segsum_wide(x0, src, dst)
    y2 = _mlp_a(x0, p1[0], p1[1], Wa1, ba1.reshape(1, H), Wa2,
                ba2.reshape(1, H), Wb1)

    p2 = _segsum_wide(y2, src, dst)
    wdem_r = jnp.pad(Wdem.reshape(N, L), ((0, NPAD - N), (0, 0)))
    feat, dem_raw = _mlp_b(y2, p2[0], p2[1], bb1.reshape(1, L), Wb2,
                           bb2.reshape(1, L), wdem_r)

    p3 = _segsum_wide(feat, src, dst)
    y4 = _mlp_c(feat, p3[0], p3[1], Wc1, bc1.reshape(1, H), Wc2,
                bc2.reshape(1, H), Wd1)

    y4f = y4.reshape(NPAD)
    p4 = _segsum_scalar(y4f, src3, dst3)

    reg8, dem = _final(
        y4f.reshape(_R8, 128),
        p4[0].reshape(_R8, 128),
        p4[1].reshape(_R8, 128),
        bd1.reshape(1, 1), Wd2.reshape(1, 1), bd2.reshape(1, 1),
        dem_raw, bdem.reshape(1, 1),
    )

    region_scores = reg8.reshape(NPAD, 1)[:N]
    return dem, region_scores


# speedup vs baseline: 1.0051x; 1.0051x over previous
import functools

import jax
import jax.numpy as jnp
from jax import lax
from jax.experimental import pallas as pl
from jax.experimental.pallas import tpu as pltpu
from jax.experimental.pallas import tpu_sc as plsc

N = 10000
E = 320000
T = 128
H = 512
L = 128

NPAD = 10240
BLK = 512
NBLK = NPAD // BLK

NC = 2
NS = 16
NW = NC * NS
EPW = E // NW
RPT = NPAD // NS
KC2 = 125
NC2 = EPW // KC2


KW = 128
NWCH = EPW // KW
TAIL = EPW - NWCH * KW
assert NWCH % 4 == 2 and TAIL % 8 == 0


@functools.partial(
    pl.kernel,
    out_type=jax.ShapeDtypeStruct((NC, NPAD, L), jnp.float32),
    mesh=plsc.VectorSubcoreMesh(core_axis_name="c", subcore_axis_name="s"),
    scratch_types=[
        pltpu.VMEM((4, KW), jnp.int32),
        pltpu.VMEM((4, KW), jnp.int32),
        pltpu.VMEM((TAIL,), jnp.int32),
        pltpu.VMEM((TAIL,), jnp.int32),
        pltpu.VMEM((KW, L), jnp.float32),
        pltpu.VMEM((KW, L), jnp.float32),
        pltpu.VMEM_SHARED((NPAD, L), jnp.float32),
        pltpu.SemaphoreType.DMA,
        pltpu.SemaphoreType.DMA,
        pltpu.SemaphoreType.DMA,
        pltpu.SemaphoreType.DMA,
        pltpu.SemaphoreType.DMA,
        pltpu.SemaphoreType.DMA,
        pltpu.SemaphoreType.DMA,
        pltpu.SemaphoreType.DMA,
        pltpu.SemaphoreType.DMA,
        pltpu.SemaphoreType.DMA,
    ],
)
def _segsum_wide(x_hbm, src_hbm, dst_hbm, out_hbm, sidx, didx, tsi, tdi,
                 rows0, rows1, acc,
                 ss0, ss1, ss2, ss3, ds0, ds1, ds2, ds3, g0, g1):
    c = lax.axis_index("c")
    s = lax.axis_index("s")
    wid = s * NC + c
    ebase = wid * EPW
    ssem = (ss0, ss1, ss2, ss3)
    dsem = (ds0, ds1, ds2, ds3)
    rows = (rows0, rows1)
    gsem = (g0, g1)

    def load_idx(j, p):
        off = ebase + j * KW
        pltpu.async_copy(src_hbm.at[pl.ds(off, KW)], sidx.at[p], ssem[p])
        pltpu.async_copy(dst_hbm.at[pl.ds(off, KW)], didx.at[p], dsem[p])

    def wait_s(p):
        pltpu.make_async_copy(src_hbm.at[pl.ds(0, KW)], sidx.at[p],
                              ssem[p]).wait()

    def start(p, b):
        pltpu.async_copy(x_hbm.at[sidx.at[p]], rows[b], gsem[b])

    def finish(p, b):
        pltpu.make_async_copy(x_hbm.at[sidx.at[p]], rows[b], gsem[b]).wait()
        pltpu.make_async_copy(dst_hbm.at[pl.ds(0, KW)], didx.at[p],
                              dsem[p]).wait()
        pltpu.sync_copy(rows[b], acc.at[didx.at[p]], add=True)

    load_idx(0, 0)
    load_idx(1, 1)
    load_idx(2, 2)
    load_idx(3, 3)

    z16 = jnp.zeros((16,), jnp.float32)

    def zero_block(i, carry):
        rows0[i // 8, pl.ds((i % 8) * 16, 16)] = z16
        return carry

    lax.fori_loop(0, KW * 8, zero_block, 0)

    row0 = s * RPT

    def zero_acc(j, carry):
        pltpu.sync_copy(rows0, acc.at[pl.ds(row0 + j * KW, KW)])
        return carry

    lax.fori_loop(0, RPT // KW, zero_acc, 0)
    plsc.subcore_barrier()

    wait_s(0)
    start(0, 0)

    def quad(jo, carry):
        j = jo * 4
        wait_s(1)
        start(1, 1)
        finish(0, 0)
        load_idx(j + 4, 0)
        wait_s(2)
        start(2, 0)
        finish(1, 1)

        @pl.when(j + 5 < NWCH)
        def _():
            load_idx(j + 5, 1)

        wait_s(3)
        start(3, 1)
        finish(2, 0)

        @pl.when(j + 6 < NWCH)
        def _():
            load_idx(j + 6, 2)

        wait_s(0)
        start(0, 0)
        finish(3, 1)

        @pl.when(j + 7 < NWCH)
        def _():
            load_idx(j + 7, 3)

        return carry

    lax.fori_loop(0, (NWCH - 2) // 4, quad, 0)
    wait_s(1)
    start(1, 1)
    finish(0, 0)
    finish(1, 1)

    toff = ebase + NWCH * KW
    pltpu.sync_copy(src_hbm.at[pl.ds(toff, TAIL)], tsi)
    pltpu.sync_copy(dst_hbm.at[pl.ds(toff, TAIL)], tdi)
    pltpu.async_copy(x_hbm.at[tsi], rows0.at[pl.ds(0, TAIL)], g0).wait()
    pltpu.sync_copy(rows0.at[pl.ds(0, TAIL)], acc.at[tdi], add=True)

    plsc.subcore_barrier()
    pltpu.sync_copy(acc.at[pl.ds(row0, RPT)], out_hbm.at[c, pl.ds(row0, RPT)])


@functools.partial(
    pl.kernel,
    out_type=jax.ShapeDtypeStruct((NC, NPAD), jnp.float32),
    mesh=plsc.VectorSubcoreMesh(core_axis_name="c", subcore_axis_name="s"),
    scratch_types=[
        pltpu.VMEM((NC2, KC2), jnp.int32),
        pltpu.VMEM((NC2, KC2), jnp.int32),
        pltpu.VMEM((KC2,), jnp.float32),
        pltpu.VMEM((KC2,), jnp.float32),
        pltpu.VMEM((RPT,), jnp.float32),
        pltpu.VMEM_SHARED((NPAD,), jnp.float32),
        pltpu.SemaphoreType.DMA,
        pltpu.SemaphoreType.DMA,
        pltpu.SemaphoreType.DMA,
    ],
)
def _segsum_scalar(x_hbm, src_hbm, dst_hbm, out_hbm, src2, dst2, rows0, rows1,
                   zv, acc, isem, sem0, sem1):
    c = lax.axis_index("c")
    s = lax.axis_index("s")
    wid = s * NC + c

    idx_cp0 = pltpu.async_copy(src_hbm.at[wid], src2, isem)
    idx_cp1 = pltpu.async_copy(dst_hbm.at[wid], dst2, isem)

    z16 = jnp.zeros((16,), jnp.float32)

    def zero_block(i, carry):
        zv[pl.ds(i * 16, 16)] = z16
        return carry

    lax.fori_loop(0, RPT // 16, zero_block, 0)

    row0 = s * RPT
    pltpu.sync_copy(zv, acc.at[pl.ds(row0, RPT)])
    idx_cp0.wait()
    idx_cp1.wait()
    plsc.subcore_barrier()

    rows = (rows0, rows1)
    sems = (sem0, sem1)

    def start(j, b):
        pltpu.async_copy(x_hbm.at[src2.at[j]], rows[b], sems[b])

    def finish(j, b):
        pltpu.make_async_copy(x_hbm.at[src2.at[j]], rows[b], sems[b]).wait()
        pltpu.sync_copy(rows[b], acc.at[dst2.at[j]], add=True)

    start(0, 0)

    def pair(jo, carry):
        j = jo * 2
        start(j + 1, 1)
        finish(j, 0)
        start(j + 2, 0)
        finish(j + 1, 1)
        return carry

    if NC2 % 2 == 0:
        lax.fori_loop(0, NC2 // 2 - 1, pair, 0)
        start(NC2 - 1, 1)
        finish(NC2 - 2, 0)
        finish(NC2 - 1, 1)
    else:
        lax.fori_loop(0, NC2 // 2, pair, 0)
        finish(NC2 - 1, 0)

    plsc.subcore_barrier()
    pltpu.sync_copy(acc.at[pl.ds(row0, RPT)], out_hbm.at[c, pl.ds(row0, RPT)])


def _dot(a, b):
    return jnp.dot(a, b, preferred_element_type=jnp.float32)


def _mlp_a_body(x_ref, p0_ref, p1_ref, wa1_ref, ba1_ref, wa2_ref, ba2_ref,
                wb1_ref, y2_ref):
    x = x_ref[...] + p0_ref[...] + p1_ref[...]
    h = jnp.maximum(_dot(x, wa1_ref[...]) + ba1_ref[...], 0.0)
    x1 = jnp.maximum(_dot(h, wa2_ref[...]) + ba2_ref[...], 0.0)
    y2_ref[...] = _dot(x1, wb1_ref[...])


_mlp_a = pl.pallas_call(
    _mlp_a_body,
    grid=(NBLK,),
    in_specs=[
        pl.BlockSpec((BLK, T), lambda i: (i, 0)),
        pl.BlockSpec((BLK, T), lambda i: (i, 0)),
        pl.BlockSpec((BLK, T), lambda i: (i, 0)),
        pl.BlockSpec((T, H), lambda i: (0, 0)),
        pl.BlockSpec((1, H), lambda i: (0, 0)),
        pl.BlockSpec((H, H), lambda i: (0, 0)),
        pl.BlockSpec((1, H), lambda i: (0, 0)),
        pl.BlockSpec((H, L), lambda i: (0, 0)),
    ],
    out_specs=pl.BlockSpec((BLK, L), lambda i: (i, 0)),
    out_shape=jax.ShapeDtypeStruct((NPAD, L), jnp.float32),
)


def _mlp_b_body(y2_ref, p0_ref, p1_ref, bb1_ref, wb2_ref, bb2_ref, wdem_ref,
                feat_ref, dem_ref):
    h = jnp.maximum(y2_ref[...] + p0_ref[...] + p1_ref[...] + bb1_ref[...], 0.0)
    feat = _dot(h, wb2_ref[...]) + bb2_ref[...]
    feat_ref[...] = feat

    @pl.when(pl.program_id(0) == 0)
    def _():
        dem_ref[...] = jnp.zeros_like(dem_ref)

    dem_ref[...] += jnp.sum(feat * wdem_ref[...]).reshape(1, 1)


_mlp_b = pl.pallas_call(
    _mlp_b_body,
    grid=(NBLK,),
    in_specs=[
        pl.BlockSpec((BLK, L), lambda i: (i, 0)),
        pl.BlockSpec((BLK, L), lambda i: (i, 0)),
        pl.BlockSpec((BLK, L), lambda i: (i, 0)),
        pl.BlockSpec((1, L), lambda i: (0, 0)),
        pl.BlockSpec((L, L), lambda i: (0, 0)),
        pl.BlockSpec((1, L), lambda i: (0, 0)),
        pl.BlockSpec((BLK, L), lambda i: (i, 0)),
    ],
    out_specs=[
        pl.BlockSpec((BLK, L), lambda i: (i, 0)),
        pl.BlockSpec((1, 1), lambda i: (0, 0)),
    ],
    out_shape=[
        jax.ShapeDtypeStruct((NPAD, L), jnp.float32),
        jax.ShapeDtypeStruct((1, 1), jnp.float32),
    ],
)


def _mlp_c_body(f_ref, p0_ref, p1_ref, wc1_ref, bc1_ref, wc2_ref, bc2_ref,
                wd1_ref, y4_ref):
    x = f_ref[...] + p0_ref[...] + p1_ref[...]
    h = jnp.maximum(_dot(x, wc1_ref[...]) + bc1_ref[...], 0.0)
    x3 = jnp.maximum(_dot(h, wc2_ref[...]) + bc2_ref[...], 0.0)
    y4_ref[...] = _dot(x3, wd1_ref[...])


_mlp_c = pl.pallas_call(
    _mlp_c_body,
    grid=(NBLK,),
    in_specs=[
        pl.BlockSpec((BLK, L), lambda i: (i, 0)),
        pl.BlockSpec((BLK, L), lambda i: (i, 0)),
        pl.BlockSpec((BLK, L), lambda i: (i, 0)),
        pl.BlockSpec((L, H), lambda i: (0, 0)),
        pl.BlockSpec((1, H), lambda i: (0, 0)),
        pl.BlockSpec((H, H), lambda i: (0, 0)),
        pl.BlockSpec((1, H), lambda i: (0, 0)),
        pl.BlockSpec((H, 1), lambda i: (0, 0)),
    ],
    out_specs=pl.BlockSpec((BLK, 1), lambda i: (i, 0)),
    out_shape=jax.ShapeDtypeStruct((NPAD, 1), jnp.float32),
)


def _final_body(y4_ref, q0_ref, q1_ref, bd1_ref, wd2_ref, bd2_ref,
                demraw_ref, bdem_ref, reg_ref, dem_ref):
    s4 = jnp.maximum(y4_ref[...] + q0_ref[...] + q1_ref[...] + bd1_ref[...], 0.0)
    reg_ref[...] = jax.nn.sigmoid(s4 * wd2_ref[...] + bd2_ref[...])
    dem_ref[...] = jax.nn.sigmoid(demraw_ref[...] + bdem_ref[...])


_R8 = NPAD // 128

_final = pl.pallas_call(
    _final_body,
    grid=(1,),
    in_specs=[
        pl.BlockSpec((_R8, 128), lambda i: (0, 0)),
        pl.BlockSpec((_R8, 128), lambda i: (0, 0)),
        pl.BlockSpec((_R8, 128), lambda i: (0, 0)),
        pl.BlockSpec((1, 1), lambda i: (0, 0)),
        pl.BlockSpec((1, 1), lambda i: (0, 0)),
        pl.BlockSpec((1, 1), lambda i: (0, 0)),
        pl.BlockSpec((1, 1), lambda i: (0, 0)),
        pl.BlockSpec((1, 1), lambda i: (0, 0)),
    ],
    out_specs=[
        pl.BlockSpec((_R8, 128), lambda i: (0, 0)),
        pl.BlockSpec((1, 1), lambda i: (0, 0)),
    ],
    out_shape=[
        jax.ShapeDtypeStruct((_R8, 128), jnp.float32),
        jax.ShapeDtypeStruct((1, 1), jnp.float32),
    ],
)


def kernel(eeg_nodes, eeg_idx, eeg_attr, Wa1, ba1, Wa2, ba2, Wb1, bb1, Wb2,
           bb2, Wc1, bc1, Wc2, bc2, Wd1, bd1, Wd2, bd2, Wdem, bdem):
    src = eeg_idx[0]
    dst = eeg_idx[1]
    src3 = src.reshape(NW, NC2, KC2)
    dst3 = dst.reshape(NW, NC2, KC2)

    x0 = jnp.pad(eeg_nodes, ((0, NPAD - N), (0, 0)))

    p1 = _segsum_wide(x0, src, dst)
    y2 = _mlp_a(x0, p1[0], p1[1], Wa1, ba1.reshape(1, H), Wa2,
                ba2.reshape(1, H), Wb1)

    p2 = _segsum_wide(y2, src, dst)
    wdem_r = jnp.pad(Wdem.reshape(N, L), ((0, NPAD - N), (0, 0)))
    feat, dem_raw = _mlp_b(y2, p2[0], p2[1], bb1.reshape(1, L), Wb2,
                           bb2.reshape(1, L), wdem_r)

    p3 = _segsum_wide(feat, src, dst)
    y4 = _mlp_c(feat, p3[0], p3[1], Wc1, bc1.reshape(1, H), Wc2,
                bc2.reshape(1, H), Wd1)

    y4f = y4.reshape(NPAD)
    p4 = _segsum_scalar(y4f, src3, dst3)

    reg8, dem = _final(
        y4f.reshape(_R8, 128),
        p4[0].reshape(_R8, 128),
        p4[1].reshape(_R8, 128),
        bd1.reshape(1, 1), Wd2.reshape(1, 1), bd2.reshape(1, 1),
        dem_raw, bdem.reshape(1, 1),
    )

    region_scores = reg8.reshape(NPAD, 1)[:N]
    return dem, region_scores
